# initial kernel scaffold (unmeasured)
import jax
import jax.numpy as jnp
from jax import lax
from jax.experimental import pallas as pl
from jax.experimental.pallas import tpu as pltpu


def kernel(
    x,
):
    def body(*refs):
        pass

    out_shape = jax.ShapeDtypeStruct(..., jnp.float32)
    return pl.pallas_call(body, out_shape=out_shape)(...)


# baseline (device time: 18460 ns/iter reference)
import jax
import jax.numpy as jnp
from jax import lax
from jax.experimental import pallas as pl
from jax.experimental.pallas import tpu as pltpu

N_DEV = 8


def kernel(x):
    m_per, n = x.shape

    def body(x_ref, out_ref, comm_ref, send_sems, recv_sems):
        my_pos = lax.axis_index("i")
        left = lax.rem(my_pos + (N_DEV - 1), N_DEV)
        right = lax.rem(my_pos + 1, N_DEV)

        barrier_sem = pltpu.get_barrier_semaphore()
        for nbr in (left, right):
            pl.semaphore_signal(
                barrier_sem, inc=1,
                device_id=(nbr,), device_id_type=pl.DeviceIdType.MESH,
            )
        pl.semaphore_wait(barrier_sem, 2)

        xv = x_ref[:, :]
        vmax = jnp.max(xv, axis=0, keepdims=True)
        rows = lax.broadcasted_iota(jnp.int32, (m_per, n), 0)
        idx_local = jnp.min(
            jnp.where(xv == vmax, rows, m_per), axis=0, keepdims=True
        )
        comm_ref[0, 0:1, :] = vmax
        comm_ref[0, 1:2, :] = (idx_local + my_pos * m_per).astype(jnp.float32)

        for h in range(N_DEV - 1):
            rdma = pltpu.make_async_remote_copy(
                src_ref=comm_ref.at[h],
                dst_ref=comm_ref.at[h + 1],
                send_sem=send_sems.at[h],
                recv_sem=recv_sems.at[h],
                device_id=(right,),
                device_id_type=pl.DeviceIdType.MESH,
            )
            rdma.start()
            rdma.wait()

        bv = comm_ref[0, 0:1, :]
        bi = comm_ref[0, 1:2, :]
        for s in range(1, N_DEV):
            v = comm_ref[s, 0:1, :]
            i = comm_ref[s, 1:2, :]
            take = (v > bv) | ((v == bv) & (i < bi))
            bv = jnp.where(take, v, bv)
            bi = jnp.where(take, i, bi)
        out_ref[0:1, :] = bv
        out_ref[1:2, :] = bi

    return pl.pallas_call(
        body,
        out_shape=jax.ShapeDtypeStruct((2, n), jnp.float32),
        in_specs=[pl.BlockSpec(memory_space=pltpu.VMEM)],
        out_specs=pl.BlockSpec(memory_space=pltpu.VMEM),
        scratch_shapes=[
            pltpu.VMEM((N_DEV, 2, n), jnp.float32),
            pltpu.SemaphoreType.DMA((N_DEV - 1,)),
            pltpu.SemaphoreType.DMA((N_DEV - 1,)),
        ],
        compiler_params=pltpu.CompilerParams(collective_id=0),
    )(x)


# device time: 11690 ns/iter; 1.5791x vs baseline; 1.5791x over previous
import jax
import jax.numpy as jnp
from jax import lax
from jax.experimental import pallas as pl
from jax.experimental.pallas import tpu as pltpu

N_DEV = 8
LOG2_N = 3


def kernel(x):
    m_per, n = x.shape

    def body(x_ref, out_ref, send_buf, recv_buf, send_sems, recv_sems):
        my_pos = lax.axis_index("i")
        partners = [my_pos ^ (1 << k) for k in range(LOG2_N)]

        barrier_sem = pltpu.get_barrier_semaphore()
        for p in partners:
            pl.semaphore_signal(
                barrier_sem, inc=1,
                device_id=(p,), device_id_type=pl.DeviceIdType.MESH,
            )
        pl.semaphore_wait(barrier_sem, LOG2_N)

        xv = x_ref[:, :]
        bv = jnp.max(xv, axis=0, keepdims=True)
        rows = lax.broadcasted_iota(jnp.int32, (m_per, n), 0)
        idx_local = jnp.min(
            jnp.where(xv == bv, rows, m_per), axis=0, keepdims=True
        )
        bi = (idx_local + my_pos * m_per).astype(jnp.float32)

        for k in range(LOG2_N):
            send_buf[k, 0:1, :] = bv
            send_buf[k, 1:2, :] = bi
            rdma = pltpu.make_async_remote_copy(
                src_ref=send_buf.at[k],
                dst_ref=recv_buf.at[k],
                send_sem=send_sems.at[k],
                recv_sem=recv_sems.at[k],
                device_id=(partners[k],),
                device_id_type=pl.DeviceIdType.MESH,
            )
            rdma.start()
            rdma.wait()
            v = recv_buf[k, 0:1, :]
            i = recv_buf[k, 1:2, :]
            take = (v > bv) | ((v == bv) & (i < bi))
            bv = jnp.where(take, v, bv)
            bi = jnp.where(take, i, bi)

        out_ref[0:1, :] = bv
        out_ref[1:2, :] = bi

    return pl.pallas_call(
        body,
        out_shape=jax.ShapeDtypeStruct((2, n), jnp.float32),
        in_specs=[pl.BlockSpec(memory_space=pltpu.VMEM)],
        out_specs=pl.BlockSpec(memory_space=pltpu.VMEM),
        scratch_shapes=[
            pltpu.VMEM((LOG2_N, 2, n), jnp.float32),
            pltpu.VMEM((LOG2_N, 2, n), jnp.float32),
            pltpu.SemaphoreType.DMA((LOG2_N,)),
            pltpu.SemaphoreType.DMA((LOG2_N,)),
        ],
        compiler_params=pltpu.CompilerParams(collective_id=0),
    )(x)


# device time: 8367 ns/iter; 2.2063x vs baseline; 1.3972x over previous
import jax
import jax.numpy as jnp
from jax import lax
from jax.experimental import pallas as pl
from jax.experimental.pallas import tpu as pltpu

N_DEV = 8


def kernel(x):
    m_per, n = x.shape

    def body(x_ref, out_ref, part_buf, send_sems, recv_sems):
        my_pos = lax.axis_index("i")

        barrier_sem = pltpu.get_barrier_semaphore()
        for j in range(1, N_DEV):
            peer = lax.rem(my_pos + j, N_DEV)
            pl.semaphore_signal(
                barrier_sem, inc=1,
                device_id=(peer,), device_id_type=pl.DeviceIdType.MESH,
            )

        xv = x_ref[:, :]
        bv = jnp.max(xv, axis=0, keepdims=True)
        rows = lax.broadcasted_iota(jnp.int32, (m_per, n), 0)
        idx_local = jnp.min(
            jnp.where(xv == bv, rows, m_per), axis=0, keepdims=True
        )
        bi = (idx_local + my_pos * m_per).astype(jnp.float32)
        part_buf[0, 0:1, :] = bv
        part_buf[0, 1:2, :] = bi

        pl.semaphore_wait(barrier_sem, N_DEV - 1)

        rdmas = []
        for j in range(1, N_DEV):
            rdma = pltpu.make_async_remote_copy(
                src_ref=part_buf.at[0],
                dst_ref=part_buf.at[N_DEV - j],
                send_sem=send_sems.at[j - 1],
                recv_sem=recv_sems.at[N_DEV - j],
                device_id=(lax.rem(my_pos + j, N_DEV),),
                device_id_type=pl.DeviceIdType.MESH,
            )
            rdma.start()
            rdmas.append(rdma)

        for s in range(1, N_DEV):
            rdmas[N_DEV - 1 - s].wait_recv()
            v = part_buf[s, 0:1, :]
            i = part_buf[s, 1:2, :]
            take = (v > bv) | ((v == bv) & (i < bi))
            bv = jnp.where(take, v, bv)
            bi = jnp.where(take, i, bi)

        out_ref[0:1, :] = bv
        out_ref[1:2, :] = bi

        for rdma in rdmas:
            rdma.wait_send()

    return pl.pallas_call(
        body,
        out_shape=jax.ShapeDtypeStruct((2, n), jnp.float32),
        in_specs=[pl.BlockSpec(memory_space=pltpu.VMEM)],
        out_specs=pl.BlockSpec(memory_space=pltpu.VMEM),
        scratch_shapes=[
            pltpu.VMEM((N_DEV, 2, n), jnp.float32),
            pltpu.SemaphoreType.DMA((N_DEV - 1,)),
            pltpu.SemaphoreType.DMA((N_DEV,)),
        ],
        compiler_params=pltpu.CompilerParams(collective_id=0),
    )(x)
